# baseline (speedup 1.0000x reference)
import functools

import jax
import jax.numpy as jnp
from jax import lax
from jax.experimental import pallas as pl
from jax.experimental.pallas import tpu as pltpu
from jax.experimental.pallas import tpu_sc as plsc

_NC = 2
_NS = 16
_NW = _NC * _NS
_CHUNK = 128


@functools.lru_cache(maxsize=None)
def _make_sc_gather(V, D, B):
    bpw = B // _NW
    hpw = bpw // 2
    n_hc = hpw // _CHUNK
    half = B // 2
    mesh = plsc.VectorSubcoreMesh(core_axis_name="c", subcore_axis_name="s")

    @functools.partial(
        pl.kernel,
        mesh=mesh,
        out_type=jax.ShapeDtypeStruct((5 * B, 2 * D), jnp.float32),
        scratch_types=[
            pltpu.VMEM((bpw,), jnp.int32),
            pltpu.VMEM((bpw, D), jnp.float32),
            pltpu.SemaphoreType.DMA,
        ],
        compiler_params=pltpu.CompilerParams(use_tc_tiling_on_sc=False),
    )
    def sc_gather(table_hbm, idx_hbm, out_hbm, idx_v, rows_v, sem):
        wid = lax.axis_index("s") * _NC + lax.axis_index("c")
        base = wid * hpw
        pltpu.sync_copy(idx_hbm.at[pl.ds(base, hpw)], idx_v.at[pl.ds(0, hpw)])
        pltpu.sync_copy(
            idx_hbm.at[pl.ds(half + base, hpw)], idx_v.at[pl.ds(hpw, hpw)]
        )
        copies = []
        for c in range(2 * n_hc):
            copies.append(
                pltpu.async_copy(
                    table_hbm.at[idx_v.at[pl.ds(c * _CHUNK, _CHUNK)]],
                    rows_v.at[pl.ds(c * _CHUNK, _CHUNK)],
                    sem,
                )
            )
        for cp in copies:
            cp.wait()
        pltpu.sync_copy(
            rows_v.at[pl.ds(0, hpw)],
            out_hbm.at[pl.ds(base, hpw), pl.ds(0, D)],
        )
        pltpu.sync_copy(
            rows_v.at[pl.ds(hpw, hpw)],
            out_hbm.at[pl.ds(base, hpw), pl.ds(D, D)],
        )

    return sc_gather


@functools.lru_cache(maxsize=None)
def _make_transpose(D, B, blk=1024):
    half = B // 2
    n = half // blk

    def body(in_hbm, out_ref, buf, sems):
        i = pl.program_id(0)
        h = pl.program_id(1)

        @pl.when(jnp.logical_and(i == 0, h == 0))
        def _():
            for k in range(n):
                pltpu.make_async_copy(
                    in_hbm.at[pl.ds(k * blk, blk), pl.ds(0, 2 * D)],
                    buf.at[k],
                    sems.at[k],
                ).start()

        @pl.when(h == 0)
        def _():
            pltpu.make_async_copy(
                in_hbm.at[pl.ds(i * blk, blk), pl.ds(0, 2 * D)],
                buf.at[i],
                sems.at[i],
            ).wait()

        @pl.when(h == 0)
        def _():
            out_ref[...] = buf[i].T[:D, :]

        @pl.when(h == 1)
        def _():
            out_ref[...] = buf[i].T[D:, :]

    def run(x):
        return pl.pallas_call(
            body,
            out_shape=jax.ShapeDtypeStruct((D, B), jnp.float32),
            grid=(n, 2),
            in_specs=[pl.BlockSpec(memory_space=pl.ANY)],
            out_specs=pl.BlockSpec((D, blk), lambda i, h: (0, h * (half // blk) + i)),
            scratch_shapes=[
                pltpu.VMEM((n, blk, 2 * D), jnp.float32),
                pltpu.SemaphoreType.DMA((n,)),
            ],
        )(x)

    return run


def kernel(speaker, embedding_table):
    idx = speaker.astype(jnp.int32)
    (B,) = idx.shape
    V, D = embedding_table.shape
    staged = _make_sc_gather(V, D, B)(embedding_table, idx)
    out_t = _make_transpose(D, B)(staged)
    return out_t.T


# speedup vs baseline: 1.0969x; 1.0969x over previous
import functools

import jax
import jax.numpy as jnp
from jax import lax
from jax.experimental import pallas as pl
from jax.experimental.pallas import tpu as pltpu
from jax.experimental.pallas import tpu_sc as plsc

_NC = 2
_NS = 16
_NW = _NC * _NS
_CHUNK = 128


@functools.lru_cache(maxsize=None)
def _make_sc_gather(V, D, B):
    bpw = B // _NW
    n_chunks = bpw // _CHUNK
    mesh = plsc.VectorSubcoreMesh(core_axis_name="c", subcore_axis_name="s")

    @functools.partial(
        pl.kernel,
        mesh=mesh,
        out_type=jax.ShapeDtypeStruct((5 * B, 2 * D), jnp.float32),
        scratch_types=[
            pltpu.VMEM((bpw,), jnp.int32),
            pltpu.VMEM((bpw, D), jnp.float32),
            pltpu.SemaphoreType.DMA,
        ],
        compiler_params=pltpu.CompilerParams(use_tc_tiling_on_sc=False),
    )
    def sc_gather(table_hbm, idx_hbm, out_hbm, idx_v, rows_v, sem):
        wid = lax.axis_index("s") * _NC + lax.axis_index("c")
        base = wid * bpw
        pltpu.sync_copy(idx_hbm.at[pl.ds(base, bpw)], idx_v)
        copies = []
        for c in range(n_chunks):
            copies.append(
                pltpu.async_copy(
                    table_hbm.at[idx_v.at[pl.ds(c * _CHUNK, _CHUNK)]],
                    rows_v.at[pl.ds(c * _CHUNK, _CHUNK)],
                    sem,
                )
            )
        for cp in copies:
            cp.wait()
        pltpu.sync_copy(rows_v, out_hbm.at[pl.ds(base, bpw), pl.ds(0, D)])

    return sc_gather


@functools.lru_cache(maxsize=None)
def _make_transpose(D, B, blk=2048):
    n = B // blk

    def body(in_hbm, out_ref, buf, sems):
        i = pl.program_id(0)

        @pl.when(i == 0)
        def _():
            for k in range(n):
                pltpu.make_async_copy(
                    in_hbm.at[pl.ds(k * blk, blk), pl.ds(0, 2 * D)],
                    buf.at[k],
                    sems.at[k],
                ).start()

        pltpu.make_async_copy(
            in_hbm.at[pl.ds(i * blk, blk), pl.ds(0, 2 * D)], buf.at[i], sems.at[i]
        ).wait()
        out_ref[...] = buf[i].T[:D, :]

    def run(x):
        return pl.pallas_call(
            body,
            out_shape=jax.ShapeDtypeStruct((D, B), jnp.float32),
            grid=(n,),
            in_specs=[pl.BlockSpec(memory_space=pl.ANY)],
            out_specs=pl.BlockSpec((D, blk), lambda i: (0, i)),
            scratch_shapes=[
                pltpu.VMEM((n, blk, 2 * D), jnp.float32),
                pltpu.SemaphoreType.DMA((n,)),
            ],
        )(x)

    return run


def kernel(speaker, embedding_table):
    idx = speaker.astype(jnp.int32)
    (B,) = idx.shape
    V, D = embedding_table.shape
    staged = _make_sc_gather(V, D, B)(embedding_table, idx)
    out_t = _make_transpose(D, B)(staged)
    return out_t.T


# speedup vs baseline: 1.0970x; 1.0001x over previous
import functools

import jax
import jax.numpy as jnp
from jax import lax
from jax.experimental import pallas as pl
from jax.experimental.pallas import tpu as pltpu
from jax.experimental.pallas import tpu_sc as plsc

_NC = 2
_NS = 16
_NW = _NC * _NS
_CHUNK = 128


@functools.lru_cache(maxsize=None)
def _make_sc_gather(V, D, B):
    bpw = B // _NW
    n_chunks = bpw // _CHUNK
    mesh = plsc.VectorSubcoreMesh(core_axis_name="c", subcore_axis_name="s")

    @functools.partial(
        pl.kernel,
        mesh=mesh,
        out_type=jax.ShapeDtypeStruct((5 * B, 2 * D), jnp.float32),
        scratch_types=[
            pltpu.VMEM((bpw,), jnp.int32),
            pltpu.VMEM((bpw, D), jnp.float32),
            pltpu.SemaphoreType.DMA,
        ],
        compiler_params=pltpu.CompilerParams(use_tc_tiling_on_sc=False),
    )
    def sc_gather(table_hbm, idx_hbm, out_hbm, idx_v, rows_v, sem):
        wid = lax.axis_index("s") * _NC + lax.axis_index("c")
        base = wid * bpw
        pltpu.sync_copy(idx_hbm.at[pl.ds(base, bpw)], idx_v)
        copies = []
        for c in range(n_chunks):
            copies.append(
                pltpu.async_copy(
                    table_hbm.at[idx_v.at[pl.ds(c * _CHUNK, _CHUNK)]],
                    rows_v.at[pl.ds(c * _CHUNK, _CHUNK)],
                    sem,
                )
            )
        for cp in copies:
            cp.wait()
        pltpu.sync_copy(rows_v, out_hbm.at[pl.ds(base, bpw), pl.ds(0, D)])

    return sc_gather


@functools.lru_cache(maxsize=None)
def _make_transpose(D, B, blk=2048):
    n = B // blk

    def body(in_hbm, out_ref, buf, sems):
        i = pl.program_id(0)

        @pl.when(i == 0)
        def _():
            for k in range(n):
                pltpu.make_async_copy(
                    in_hbm.at[pl.ds(k * blk, blk)],
                    buf.at[k],
                    sems.at[k],
                ).start()

        pltpu.make_async_copy(
            in_hbm.at[pl.ds(i * blk, blk)], buf.at[i], sems.at[i]
        ).wait()
        out_ref[...] = buf[i].T[:D, :]

    def run(x):
        return pl.pallas_call(
            body,
            out_shape=jax.ShapeDtypeStruct((D, B), jnp.float32),
            grid=(n,),
            in_specs=[pl.BlockSpec(memory_space=pl.ANY)],
            out_specs=pl.BlockSpec((D, blk), lambda i: (0, i)),
            scratch_shapes=[
                pltpu.VMEM((n, blk, 2 * D), jnp.float32),
                pltpu.SemaphoreType.DMA((n,)),
            ],
        )(x)

    return run


def kernel(speaker, embedding_table):
    idx = speaker.astype(jnp.int32)
    (B,) = idx.shape
    V, D = embedding_table.shape
    staged = _make_sc_gather(V, D, B)(embedding_table, idx)
    out_t = _make_transpose(D, B)(staged)
    return out_t.T


# speedup vs baseline: 1.1346x; 1.0343x over previous
import functools

import jax
import jax.numpy as jnp
from jax import lax
from jax.experimental import pallas as pl
from jax.experimental.pallas import tpu as pltpu
from jax.experimental.pallas import tpu_sc as plsc

_NC = 2
_NS = 16
_NW = _NC * _NS
_CHUNK = 128


@functools.lru_cache(maxsize=None)
def _make_sc_gather_tiled(V, D, B):
    bpw = B // _NW
    n_chunks = bpw // _CHUNK
    n_tr = D // 8
    n_tc = B // 128
    w_tc = bpw // 128
    wpan = w_tc * 1024
    mesh = plsc.VectorSubcoreMesh(core_axis_name="c", subcore_axis_name="s")

    @functools.partial(
        pl.kernel,
        mesh=mesh,
        out_type=jax.ShapeDtypeStruct((n_tr, n_tc, 1024), jnp.float32),
        scratch_types=[
            pltpu.VMEM((bpw,), jnp.int32),
            pltpu.VMEM((bpw, D), jnp.float32),
            pltpu.VMEM((n_tr * wpan,), jnp.float32),
            pltpu.SemaphoreType.DMA,
            pltpu.SemaphoreType.DMA,
        ],
        compiler_params=pltpu.CompilerParams(
            use_tc_tiling_on_sc=False, needs_layout_passes=False
        ),
    )
    def sc_gather_t(table_hbm, idx_hbm, out_hbm, idx_v, rows_v, panel, gsem, wsem):
        wid = lax.axis_index("s") * _NC + lax.axis_index("c")
        base = wid * bpw
        pltpu.sync_copy(idx_hbm.at[pl.ds(base, bpw)], idx_v)
        for c in range(n_chunks):
            pltpu.async_copy(
                table_hbm.at[idx_v.at[pl.ds(c * _CHUNK, _CHUNK)]],
                rows_v.at[pl.ds(c * _CHUNK, _CHUNK)],
                gsem,
            )
        lane = lax.iota(jnp.int32, 16)
        diag = [(lane + i) % 16 for i in range(16)]
        svec = [(dg // 8) * wpan + (dg % 8) * 128 + lane for dg in diag]

        def t_body(t, _):
            jg = lax.rem(t, _CHUNK // 16)

            @pl.when(jg == 0)
            def _():
                pltpu.make_async_copy(
                    table_hbm.at[idx_v.at[pl.ds(0, _CHUNK)]],
                    rows_v.at[pl.ds(0, _CHUNK)],
                    gsem,
                ).wait()

            j_vec = t * 16 + lane
            off = lax.div(t, _CHUNK // 16) * 1024 + jg * 16
            for cg in range(D // 16):
                off_cg = off + cg * (2 * wpan)
                for i in range(16):
                    v = plsc.load_gather(rows_v, [j_vec, diag[i] + cg * 16])
                    plsc.store_scatter(panel, [svec[i] + off_cg], v)

            for c_s in range(n_chunks):
                @pl.when(t == c_s * (_CHUNK // 16) + (_CHUNK // 16 - 1))
                def _(c_s=c_s):
                    for tr in range(n_tr):
                        pltpu.async_copy(
                            panel.at[pl.ds(tr * wpan + c_s * 1024, 1024)],
                            out_hbm.at[tr, wid * w_tc + c_s],
                            wsem,
                        )
            return 0

        lax.fori_loop(0, n_chunks * (_CHUNK // 16), t_body, 0)
        for c in range(n_chunks):
            for tr in range(n_tr):
                pltpu.make_async_copy(
                    panel.at[pl.ds(tr * wpan + c * 1024, 1024)],
                    out_hbm.at[tr, wid * w_tc + c],
                    wsem,
                ).wait()

    return sc_gather_t


def kernel(speaker, embedding_table):
    idx = speaker.astype(jnp.int32)
    (B,) = idx.shape
    V, D = embedding_table.shape
    x = _make_sc_gather_tiled(V, D, B)(embedding_table, idx)
    out_t = x.reshape(D // 8, B // 128, 8, 128).transpose(0, 2, 1, 3).reshape(D, B)
    return out_t.T
